# baseline (speedup 1.0000x reference)
import functools

import jax
import jax.numpy as jnp
from jax import lax
from jax.experimental import pallas as pl
from jax.experimental.pallas import tpu as pltpu
from jax.experimental.pallas import tpu_sc as plsc

B = 4096
M = 50
MP = 64
D = 64

NC = 2
NS = 16
NW = NC * NS

ROWS = B * MP
RPW = ROWS // NW
CH = 128
NCHUNK = RPW // CH
BPW = B // NW


def _sc_gather(mids2d, item_ids, gro_ids, u2e_w, v2e_w, g2e_w):
    mesh = plsc.VectorSubcoreMesh(core_axis_name="c", subcore_axis_name="s")

    @functools.partial(
        pl.kernel,
        mesh=mesh,
        out_type=[
            jax.ShapeDtypeStruct((ROWS, D), jnp.float32),
            jax.ShapeDtypeStruct((B, D), jnp.float32),
            jax.ShapeDtypeStruct((B, D), jnp.float32),
        ],
        scratch_types=[
            pltpu.VMEM((NCHUNK, CH), jnp.int32),
            pltpu.VMEM((CH, D), jnp.float32),
            pltpu.VMEM((BPW,), jnp.int32),
            pltpu.VMEM((BPW, D), jnp.float32),
            pltpu.SemaphoreType.DMA,
        ],
        compiler_params=pltpu.CompilerParams(use_tc_tiling_on_sc=False),
    )
    def k(mids_hbm, iids_hbm, gids_hbm, u2e_hbm, v2e_hbm, g2e_hbm,
          memb_out, item_out, group_out,
          idx_v, buf_v, sid_v, rows_v, sem):
        wid = lax.axis_index("s") * NC + lax.axis_index("c")
        base = wid * RPW

        pltpu.sync_copy(mids_hbm.at[pl.ds(wid * NCHUNK, NCHUNK)], idx_v)

        def body(j, _):
            pltpu.async_copy(u2e_hbm.at[idx_v.at[j]], buf_v, sem).wait()
            pltpu.sync_copy(buf_v, memb_out.at[pl.ds(base + j * CH, CH)])
            return 0

        lax.fori_loop(0, NCHUNK, body, 0)

        sbase = wid * BPW
        pltpu.sync_copy(iids_hbm.at[pl.ds(sbase, BPW)], sid_v)
        pltpu.async_copy(v2e_hbm.at[sid_v], rows_v, sem).wait()
        pltpu.sync_copy(rows_v, item_out.at[pl.ds(sbase, BPW)])

        pltpu.sync_copy(gids_hbm.at[pl.ds(sbase, BPW)], sid_v)
        pltpu.async_copy(g2e_hbm.at[sid_v], rows_v, sem).wait()
        pltpu.sync_copy(rows_v, group_out.at[pl.ds(sbase, BPW)])

    return k(mids2d, item_ids, gro_ids, u2e_w, v2e_w, g2e_w)


BG = 8
GRID = B // BG


def _tc_attn_body(emb_ref, maskf_ref, mask_ref, item_ref, group_ref,
                  wq_ref, bq_ref, wk_ref, bk_ref, wv_ref, bv_ref,
                  w1a_ref, w1b_ref, b1_ref, w2_ref, b2_ref, out_ref):
    maskf = maskf_ref[:]
    emb = jnp.where(maskf > 0.0, emb_ref[:], 0.0)
    q = (jnp.dot(emb, wq_ref[:], preferred_element_type=jnp.float32)
         + bq_ref[:]) * maskf
    k = (jnp.dot(emb, wk_ref[:], preferred_element_type=jnp.float32)
         + bk_ref[:]) * maskf
    v = jnp.dot(emb, wv_ref[:], preferred_element_type=jnp.float32) + bv_ref[:]

    rows = []
    for g in range(BG):
        s0, s1 = g * MP, (g + 1) * MP
        qg, kg, vg = q[s0:s1], k[s0:s1], v[s0:s1]
        eg = emb[s0:s1]
        mrow = maskf[s0:s1]
        mcol = mask_ref[g:g + 1, :]
        energy = lax.dot_general(qg, kg, (((1,), (1,)), ((), ())),
                                 preferred_element_type=jnp.float32)
        energy = jnp.clip(energy, -50.0, 50.0)
        eexp = jnp.exp(energy) * mcol
        attn = eexp / jnp.sum(eexp, axis=1, keepdims=True)
        mo = jnp.dot(attn, vg, preferred_element_type=jnp.float32)
        overall = 0.5 * (mo * mrow) + 0.5 * eg
        ipart = jnp.dot(item_ref[g:g + 1, :], w1b_ref[:],
                        preferred_element_type=jnp.float32)
        h = jnp.maximum(
            jnp.dot(overall, w1a_ref[:], preferred_element_type=jnp.float32)
            + mrow * ipart + b1_ref[:], 0.0)
        a = jnp.dot(h, w2_ref[:], preferred_element_type=jnp.float32) + b2_ref[:]
        a = jnp.clip(a, -50.0, 50.0)
        aexp = jnp.exp(a) * mrow
        w = aexp / jnp.sum(aexp)
        pooled = jnp.sum(w * overall, axis=0, keepdims=True)
        rows.append(0.5 * pooled + 0.5 * group_ref[g:g + 1, :])
    out_ref[:] = jnp.concatenate(rows, axis=0)


def _tc_attn(emb_flat, maskf, mask2d, item_emb, group_emb,
             Wq, bq, Wk, bk, Wv, bv, W1a, W1b, b1, W2, b2):
    full = lambda shape: pl.BlockSpec(shape, lambda i: (0, 0))
    return pl.pallas_call(
        _tc_attn_body,
        grid=(GRID,),
        in_specs=[
            pl.BlockSpec((BG * MP, D), lambda i: (i, 0)),
            pl.BlockSpec((BG * MP, 1), lambda i: (i, 0)),
            pl.BlockSpec((BG, MP), lambda i: (i, 0)),
            pl.BlockSpec((BG, D), lambda i: (i, 0)),
            pl.BlockSpec((BG, D), lambda i: (i, 0)),
            full((D, D)), full((1, D)),
            full((D, D)), full((1, D)),
            full((D, D)), full((1, D)),
            full((D, 16)), full((D, 16)), full((1, 16)),
            full((16, 1)), full((1, 1)),
        ],
        out_specs=pl.BlockSpec((BG, D), lambda i: (i, 0)),
        out_shape=jax.ShapeDtypeStruct((B, D), jnp.float32),
    )(emb_flat, maskf, mask2d, item_emb, group_emb,
      Wq, bq, Wk, bk, Wv, bv, W1a, W1b, b1, W2, b2)


def kernel(gro_inputs, item_inputs, menb_ids, mask, u2e_w, v2e_w, g2e_w,
           Wq, bq, Wk, bk, Wv, bv, W1, b1, W2, b2):
    mids_p = jnp.pad(menb_ids, ((0, 0), (0, MP - M)))
    mids2d = mids_p.reshape(ROWS // CH, CH)
    memb_flat, item_emb, group_emb = _sc_gather(
        mids2d, item_inputs, gro_inputs, u2e_w, v2e_w, g2e_w)

    mask_p = jnp.pad(mask, ((0, 0), (0, MP - M)))
    maskf = mask_p.reshape(B * MP, 1)
    return _tc_attn(memb_flat, maskf, mask_p, item_emb, group_emb,
                    Wq, bq.reshape(1, D), Wk, bk.reshape(1, D),
                    Wv, bv.reshape(1, D),
                    W1[:D], W1[D:], b1.reshape(1, 16),
                    W2, b2.reshape(1, 1))
